# baseline (speedup 1.0000x reference)
import dataclasses
import functools

import jax
import jax.numpy as jnp
import numpy as np
from jax import lax
from jax.experimental import pallas as pl
from jax.experimental.pallas import tpu as pltpu
from jax.experimental.pallas import tpu_sc as plsc


def _positional_encoding(length: int, depth: int) -> np.ndarray:
    half = depth // 2
    positions = np.arange(length)[:, np.newaxis]
    depths = np.arange(half)[np.newaxis, :] / half
    angle_rates = 1.0 / (10000.0 ** depths)
    angle_rads = positions * angle_rates
    return np.concatenate(
        [np.sin(angle_rads), np.cos(angle_rads)], axis=-1
    ).astype(np.float32)


_NC, _NS, _L = 2, 16, 16
_NW = _NC * _NS
_W = 128
_CB, _CS = 16, 8


def kernel(x, table):
    B, S = x.shape
    V, D = table.shape
    scale = float(np.sqrt(float(D)))
    pe = jnp.asarray(_positional_encoding(S, D))

    assert B % _CB == 0 and S % _CS == 0 and D % _L == 0
    n_chunks = (B // _CB) * (S // _CS)
    assert n_chunks % _NW == 0
    per_w = n_chunks // _NW
    assert per_w >= 4 and per_w % 3 == 2
    n_sgrp = S // _CS
    bsub_per_w = per_w // n_sgrp
    rows_per_w = bsub_per_w * _CB

    xi = x.astype(jnp.int32)

    mesh = plsc.VectorSubcoreMesh(core_axis_name="c", subcore_axis_name="s")

    cp = pltpu.CompilerParams()
    if "needs_layout_passes" in pltpu.CompilerParams.__dataclass_fields__:
        cp = dataclasses.replace(cp, needs_layout_passes=False)

    @functools.partial(
        pl.kernel,
        mesh=mesh,
        compiler_params=cp,
        out_type=jax.ShapeDtypeStruct((B, S, D), jnp.float32),
        scratch_types=[
            pltpu.VMEM((S, D), jnp.float32),
            pltpu.VMEM((rows_per_w, S), jnp.int32),
            pltpu.VMEM((per_w, _W), jnp.int32),
            pltpu.VMEM((_W, D), jnp.float32),
            pltpu.VMEM((_W, D), jnp.float32),
            pltpu.VMEM((_W, D), jnp.float32),
            pltpu.VMEM((_W, D), jnp.float32),
            pltpu.SemaphoreType.DMA,
            pltpu.SemaphoreType.DMA,
            pltpu.SemaphoreType.DMA,
            pltpu.SemaphoreType.DMA,
            pltpu.SemaphoreType.DMA,
            pltpu.SemaphoreType.DMA,
            pltpu.SemaphoreType.DMA,
            pltpu.SemaphoreType.DMA,
        ],
    )
    def k(x_hbm, table_hbm, pe_hbm, out_hbm,
          pe_v, xb_v, idx_v, r0, r1, r2, r3,
          g0, g1, g2, g3, o0, o1, o2, o3):
        rows = (r0, r1, r2, r3)
        gsem = (g0, g1, g2, g3)
        osem = (o0, o1, o2, o3)
        wid = lax.axis_index("s") * _NC + lax.axis_index("c")
        b_lo = pl.multiple_of(wid * rows_per_w, rows_per_w)

        pltpu.sync_copy(pe_hbm, pe_v)
        pltpu.sync_copy(x_hbm.at[pl.ds(b_lo, rows_per_w)], xb_v)

        iota = lax.iota(jnp.int32, _L)
        bv = lax.shift_right_logical(iota, 3)
        sv = lax.bitwise_and(iota, 7)

        @pl.loop(0, per_w)
        def _(t):
            m = t // n_sgrp
            kk = t % n_sgrp
            for j in range(_W // _L):
                ridx = bv + (m * _CB + 2 * j)
                cidx = sv + kk * _CS
                vals = plsc.load_gather(xb_v, [ridx, cidx])
                idx_v[t, pl.ds(_L * j, _L)] = vals

        def gather(t, b):
            return pltpu.make_async_copy(
                table_hbm.at[idx_v.at[t]], rows[b], gsem[b])

        def out_slot(t):
            m = t // n_sgrp
            kk = t % n_sgrp
            return out_hbm.at[
                pl.ds(b_lo + m * _CB, _CB),
                pl.ds(pl.multiple_of(kk * _CS, _CS), _CS),
                slice(None),
            ]

        def writeback(t, b):
            return pltpu.make_async_copy(
                rows[b].reshape(_CB, _CS, D), out_slot(t), osem[b])

        def compute(t, b):
            kk = t % n_sgrp
            r = rows[b]
            for sp in range(_CS):
                pe_regs = [
                    pe_v[kk * _CS + sp, pl.ds(cc * _L, _L)]
                    for cc in range(D // _L)
                ]

                @pl.loop(0, _CB)
                def _(bp):
                    i = bp * _CS + sp
                    for cc in range(D // _L):
                        sl = pl.ds(cc * _L, _L)
                        r[i, sl] = r[i, sl] * scale + pe_regs[cc]

        def body(t, b, first=False, last_iter=False):
            pb = (b + 3) % 4

            def _advance():
                writeback(t - 1, pb).wait()
                gather(t + 3, pb).start()

            if first:
                pl.when(t >= 1)(lambda: writeback(t - 1, pb).wait())
                gather(t + 3, pb).start()
            elif last_iter:
                writeback(t - 1, pb).wait()
                pl.when(t + 3 < per_w)(lambda: gather(t + 3, pb).start())
            else:
                _advance()

            gather(t, b).wait()
            compute(t, b)
            writeback(t, b).start()

        gather(0, 0).start()
        gather(1, 1).start()
        gather(2, 2).start()

        @pl.loop(0, per_w - 2, step=4)
        def _(t):
            body(t, 0, first=True)
            body(t + 1, 1)
            body(t + 2, 2)
            body(t + 3, 3, last_iter=True)

        body(per_w - 2, (per_w - 2) % 4, last_iter=True)
        body(per_w - 1, (per_w - 1) % 4, last_iter=True)
        writeback(per_w - 1, (per_w - 1) % 4).wait()

    return k(xi, table, pe)


# speedup vs baseline: 1.0274x; 1.0274x over previous
import dataclasses
import functools

import jax
import jax.numpy as jnp
import numpy as np
from jax import lax
from jax.experimental import pallas as pl
from jax.experimental.pallas import tpu as pltpu
from jax.experimental.pallas import tpu_sc as plsc


def _positional_encoding(length: int, depth: int) -> np.ndarray:
    half = depth // 2
    positions = np.arange(length)[:, np.newaxis]
    depths = np.arange(half)[np.newaxis, :] / half
    angle_rates = 1.0 / (10000.0 ** depths)
    angle_rads = positions * angle_rates
    return np.concatenate(
        [np.sin(angle_rads), np.cos(angle_rads)], axis=-1
    ).astype(np.float32)


_NC, _NS, _L = 2, 16, 16
_NW = _NC * _NS
_CB, _CS = 4, 40
_CR = _CB * _CS
_G = 80
_NBUF = 4


def kernel(x, table):
    B, S = x.shape
    V, D = table.shape
    scale = float(np.sqrt(float(D)))
    pe = jnp.asarray(_positional_encoding(S, D))

    assert B % _CB == 0 and S % _CS == 0 and D % _L == 0 and _CS % 8 == 0
    n_chunks = (B // _CB) * (S // _CS)
    assert n_chunks % _NW == 0
    per_w = n_chunks // _NW
    assert per_w % _NBUF == 0 and per_w >= 2 * _NBUF
    n_sgrp = S // _CS
    bsub_per_w = per_w // n_sgrp
    rows_per_w = bsub_per_w * _CB
    n_grp = _CR // _L
    grp_per_g = _G // _L

    xi = x.astype(jnp.int32)

    mesh = plsc.VectorSubcoreMesh(core_axis_name="c", subcore_axis_name="s")

    cp = pltpu.CompilerParams()
    if "needs_layout_passes" in pltpu.CompilerParams.__dataclass_fields__:
        cp = dataclasses.replace(cp, needs_layout_passes=False)

    @functools.partial(
        pl.kernel,
        mesh=mesh,
        compiler_params=cp,
        out_type=jax.ShapeDtypeStruct((B, S, D), jnp.float32),
        scratch_types=[
            pltpu.VMEM((S, D), jnp.float32),
            pltpu.VMEM((rows_per_w, S), jnp.int32),
            pltpu.VMEM((per_w, 2, _G), jnp.int32),
        ]
        + [pltpu.VMEM((_CR, D), jnp.float32) for _ in range(_NBUF)]
        + [pltpu.SemaphoreType.DMA for _ in range(2 * _NBUF)],
    )
    def k(x_hbm, table_hbm, pe_hbm, out_hbm, pe_v, xb_v, idx_v, *bufs):
        rows = bufs[:_NBUF]
        gsem = bufs[_NBUF:2 * _NBUF]
        osem = bufs[2 * _NBUF:]
        wid = lax.axis_index("s") * _NC + lax.axis_index("c")
        b_lo = pl.multiple_of(wid * rows_per_w, rows_per_w)

        pltpu.sync_copy(pe_hbm, pe_v)
        pltpu.sync_copy(x_hbm.at[pl.ds(b_lo, rows_per_w)], xb_v)

        iota = lax.iota(jnp.int32, _L)

        @pl.loop(0, per_w)
        def _(t):
            m = t // n_sgrp
            kk = t % n_sgrp
            for j in range(n_grp):
                rv = iota + (_L * j)
                bv = rv // _CS
                sv = rv - bv * _CS
                vals = plsc.load_gather(
                    xb_v, [bv + m * _CB, sv + kk * _CS])
                idx_v[t, j // grp_per_g,
                      pl.ds((j % grp_per_g) * _L, _L)] = vals

        def gather_start(t, b):
            for g in range(2):
                pltpu.make_async_copy(
                    table_hbm.at[idx_v.at[t, g]],
                    rows[b].at[pl.ds(g * _G, _G)],
                    gsem[b],
                ).start()

        def gather_wait(t, b):
            for g in range(2):
                pltpu.make_async_copy(
                    table_hbm.at[idx_v.at[t, g]],
                    rows[b].at[pl.ds(g * _G, _G)],
                    gsem[b],
                ).wait()

        def out_slot(t):
            m = t // n_sgrp
            kk = t % n_sgrp
            return out_hbm.at[
                pl.ds(b_lo + m * _CB, _CB),
                pl.ds(pl.multiple_of(kk * _CS, _CS), _CS),
                slice(None),
            ]

        def writeback(t, b):
            return pltpu.make_async_copy(
                rows[b].reshape(_CB, _CS, D), out_slot(t), osem[b])

        def compute(t, b):
            kk = t % n_sgrp
            r = rows[b]

            @pl.loop(0, _CS)
            def _(sp):
                pe_regs = [
                    pe_v[kk * _CS + sp, pl.ds(cc * _L, _L)]
                    for cc in range(D // _L)
                ]
                for bp in range(_CB):
                    i = bp * _CS + sp
                    for cc in range(D // _L):
                        sl = pl.ds(cc * _L, _L)
                        r[i, sl] = r[i, sl] * scale + pe_regs[cc]

        def body(t, b):
            pb = (b + 3) % _NBUF
            pl.when(t >= 1)(lambda: writeback(t - 1, pb).wait())
            pl.when(t + 3 < per_w)(lambda: gather_start(t + 3, pb))
            gather_wait(t, b)
            compute(t, b)
            writeback(t, b).start()

        gather_start(0, 0)
        gather_start(1, 1)
        gather_start(2, 2)

        @pl.loop(0, per_w, step=_NBUF)
        def _(t):
            for i in range(_NBUF):
                body(t + i, i)

        writeback(per_w - 1, (per_w - 1) % _NBUF).wait()

    return k(xi, table, pe)


# speedup vs baseline: 1.0647x; 1.0364x over previous
import dataclasses
import functools

import jax
import jax.numpy as jnp
import numpy as np
from jax import lax
from jax.experimental import pallas as pl
from jax.experimental.pallas import tpu as pltpu
from jax.experimental.pallas import tpu_sc as plsc


def _positional_encoding(length: int, depth: int) -> np.ndarray:
    half = depth // 2
    positions = np.arange(length)[:, np.newaxis]
    depths = np.arange(half)[np.newaxis, :] / half
    angle_rates = 1.0 / (10000.0 ** depths)
    angle_rads = positions * angle_rates
    return np.concatenate(
        [np.sin(angle_rads), np.cos(angle_rads)], axis=-1
    ).astype(np.float32)


_NC, _NS, _L = 2, 16, 16
_NW = _NC * _NS
_CB, _CS = 4, 40
_CR = _CB * _CS
_G = 80
_NBUF = 4


def kernel(x, table):
    B, S = x.shape
    V, D = table.shape
    scale = float(np.sqrt(float(D)))
    pe = jnp.asarray(_positional_encoding(S, D))

    assert B % _CB == 0 and S % _CS == 0 and D % _L == 0 and _CS % 8 == 0
    n_chunks = (B // _CB) * (S // _CS)
    assert n_chunks % _NW == 0
    per_w = n_chunks // _NW
    assert per_w % _NBUF == 0 and per_w >= 2 * _NBUF
    n_sgrp = S // _CS
    bsub_per_w = per_w // n_sgrp
    rows_per_w = bsub_per_w * _CB
    n_grp = _CR // _L
    grp_per_g = _G // _L

    xi = x.astype(jnp.int32)

    mesh = plsc.VectorSubcoreMesh(core_axis_name="c", subcore_axis_name="s")

    cp = pltpu.CompilerParams()
    if "needs_layout_passes" in pltpu.CompilerParams.__dataclass_fields__:
        cp = dataclasses.replace(cp, needs_layout_passes=False)

    @functools.partial(
        pl.kernel,
        mesh=mesh,
        compiler_params=cp,
        out_type=jax.ShapeDtypeStruct((B, S, D), jnp.float32),
        scratch_types=[
            pltpu.VMEM((S, D), jnp.float32),
            pltpu.VMEM((rows_per_w, S), jnp.int32),
            pltpu.VMEM((per_w, 2, _G), jnp.int32),
        ]
        + [pltpu.VMEM((_CR, D), jnp.float32) for _ in range(_NBUF)]
        + [pltpu.SemaphoreType.DMA for _ in range(2 * _NBUF)],
    )
    def k(x_hbm, table_hbm, pe_hbm, out_hbm, pe_v, xb_v, idx_v, *bufs):
        rows = bufs[:_NBUF]
        gsem = bufs[_NBUF:2 * _NBUF]
        osem = bufs[2 * _NBUF:]
        wid = lax.axis_index("s") * _NC + lax.axis_index("c")
        b_lo = pl.multiple_of(wid * rows_per_w, rows_per_w)

        pe_copy = pltpu.make_async_copy(pe_hbm, pe_v, osem[0])
        pltpu.sync_copy(x_hbm.at[pl.ds(b_lo, rows_per_w)], xb_v)

        iota = lax.iota(jnp.int32, _L)
        bvs, svs = [], []
        for j in range(n_grp):
            rv = iota + (_L * j)
            bv = rv // _CS
            bvs.append(bv)
            svs.append(rv - bv * _CS)

        def build_idx(t):
            m = t // n_sgrp
            kk = t % n_sgrp
            for j in range(n_grp):
                vals = plsc.load_gather(
                    xb_v, [bvs[j] + m * _CB, svs[j] + kk * _CS])
                idx_v[t, j // grp_per_g,
                      pl.ds((j % grp_per_g) * _L, _L)] = vals

        def gather_start(t, b):
            for g in range(2):
                pltpu.make_async_copy(
                    table_hbm.at[idx_v.at[t, g]],
                    rows[b].at[pl.ds(g * _G, _G)],
                    gsem[b],
                ).start()

        def gather_wait(t, b):
            for g in range(2):
                pltpu.make_async_copy(
                    table_hbm.at[idx_v.at[t, g]],
                    rows[b].at[pl.ds(g * _G, _G)],
                    gsem[b],
                ).wait()

        def out_slot(t):
            m = t // n_sgrp
            kk = t % n_sgrp
            return out_hbm.at[
                pl.ds(b_lo + m * _CB, _CB),
                pl.ds(pl.multiple_of(kk * _CS, _CS), _CS),
                slice(None),
            ]

        def writeback(t, b):
            return pltpu.make_async_copy(
                rows[b].reshape(_CB, _CS, D), out_slot(t), osem[b])

        def compute(t, b):
            kk = t % n_sgrp
            r = rows[b]

            @pl.loop(0, _CS)
            def _(sp):
                pe_regs = [
                    pe_v[kk * _CS + sp, pl.ds(cc * _L, _L)]
                    for cc in range(D // _L)
                ]
                for bp in range(_CB):
                    i = bp * _CS + sp
                    for cc in range(D // _L):
                        sl = pl.ds(cc * _L, _L)
                        r[i, sl] = r[i, sl] * scale + pe_regs[cc]

        def body(t, b):
            pb = (b + 3) % _NBUF
            pl.when(t >= 1)(lambda: writeback(t - 1, pb).wait())
            pl.when(t + 3 < per_w)(lambda: gather_start(t + 3, pb))
            gather_wait(t, b)
            compute(t, b)
            writeback(t, b).start()

        for t0 in range(3):
            build_idx(t0)
            gather_start(t0, t0)
        pe_copy.start()

        @pl.loop(3, per_w)
        def _(t):
            build_idx(t)

        pe_copy.wait()

        @pl.loop(0, per_w, step=_NBUF)
        def _(t):
            for i in range(_NBUF):
                body(t + i, i)

        writeback(per_w - 1, (per_w - 1) % _NBUF).wait()

    return k(xi, table, pe)
